# Initial kernel scaffold; baseline (speedup 1.0000x reference)
import jax
import jax.numpy as jnp
from jax.experimental import pallas as pl


def kernel(index, targets, table):
    raise NotImplementedError("write your pallas kernel here")


# speedup vs baseline: 1.9711x; 1.9711x over previous
import functools

import jax
import jax.numpy as jnp
from jax import lax
from jax.experimental import pallas as pl
from jax.experimental.pallas import tpu as pltpu
from jax.experimental.pallas import tpu_sc as plsc


def _lse_block(x_ref, o_ref):
    x = x_ref[...]
    m = jnp.max(x, axis=1)
    s = jnp.sum(jnp.exp(x - m[:, None]), axis=1)
    o_ref[0, 0, :] = m + jnp.log(s)


def _row_lse(table):
    v, c = table.shape
    rows = 256
    grid = v // rows
    out = pl.pallas_call(
        _lse_block,
        grid=(grid,),
        in_specs=[pl.BlockSpec((rows, c), lambda i: (i, 0))],
        out_specs=pl.BlockSpec((1, 1, rows), lambda i: (i, 0, 0)),
        out_shape=jax.ShapeDtypeStruct((grid, 1, rows), jnp.float32),
    )(table)
    return out.reshape(-1)


def _make_sc_gather(n, vocab, c, nc, ns, lanes):
    nw = nc * ns
    npw = n // nw
    ch = lanes
    nch = npw // ch
    mesh = plsc.VectorSubcoreMesh(core_axis_name="c", subcore_axis_name="s")

    gch = 128
    ngch = npw // gch

    @functools.partial(
        pl.kernel,
        mesh=mesh,
        out_type=[
            jax.ShapeDtypeStruct((n, c), jnp.float32),
            jax.ShapeDtypeStruct((nw, lanes), jnp.float32),
        ],
        scratch_types=[
            pltpu.VMEM((npw,), jnp.int32),
            pltpu.VMEM((npw,), jnp.int32),
            pltpu.VMEM((npw,), jnp.int32),
            pltpu.VMEM((npw,), jnp.float32),
            pltpu.VMEM((npw,), jnp.float32),
            pltpu.VMEM((ch, c), jnp.float32),
            pltpu.VMEM((lanes,), jnp.float32),
            pltpu.SemaphoreType.DMA,
            pltpu.SemaphoreType.DMA,
        ],
    )
    def sc_kernel(idx_hbm, tgt_hbm, table_hbm, tflat_hbm, lse_hbm,
                  out_hbm, part_hbm,
                  idx_v, tgt_v, fidx_v, tval_v, lsev_v, rows_v, acc_v,
                  sem, sem2):
        cid = lax.axis_index("c")
        sid = lax.axis_index("s")
        wid = sid * nc + cid
        base = wid * npw
        pltpu.sync_copy(idx_hbm.at[pl.ds(base, npw)], idx_v)
        pltpu.sync_copy(tgt_hbm.at[pl.ds(base, npw)], tgt_v)

        def fbody(ci, _):
            off = pl.multiple_of(ci * lanes, 8)
            idx16 = idx_v[pl.ds(off, lanes)]
            tgt16 = tgt_v[pl.ds(off, lanes)]
            fidx_v[pl.ds(off, lanes)] = idx16 * c + tgt16
            return 0

        lax.fori_loop(0, npw // lanes, fbody, 0)

        for j in range(ngch):
            pltpu.async_copy(
                tflat_hbm.at[fidx_v.at[pl.ds(j * gch, gch)]],
                tval_v.at[pl.ds(j * gch, gch)], sem2)
            pltpu.async_copy(
                lse_hbm.at[idx_v.at[pl.ds(j * gch, gch)]],
                lsev_v.at[pl.ds(j * gch, gch)], sem2)
        for j in range(ngch):
            pltpu.make_async_copy(
                tflat_hbm.at[fidx_v.at[pl.ds(j * gch, gch)]],
                tval_v.at[pl.ds(j * gch, gch)], sem2).wait()
            pltpu.make_async_copy(
                lse_hbm.at[idx_v.at[pl.ds(j * gch, gch)]],
                lsev_v.at[pl.ds(j * gch, gch)], sem2).wait()

        def body(ci, acc):
            off = pl.multiple_of(ci * ch, 8)
            idx16 = idx_v[pl.ds(off, ch)]
            pltpu.async_copy(table_hbm.at[idx16], rows_v, sem).wait()
            pltpu.sync_copy(rows_v, out_hbm.at[pl.ds(base + off, ch)])
            lse16 = lsev_v[pl.ds(off, ch)]
            tval16 = tval_v[pl.ds(off, ch)]
            return acc + (lse16 - tval16)

        acc = lax.fori_loop(0, nch, body, jnp.zeros((lanes,), jnp.float32))
        acc_v[...] = acc
        pltpu.sync_copy(acc_v, part_hbm.at[wid])

    return sc_kernel


def kernel(index, targets, table):
    b, t = index.shape
    vocab, c = table.shape
    n = b * t
    idx = index.reshape(-1).astype(jnp.int32)
    tgt = targets.reshape(-1).astype(jnp.int32)
    lse = _row_lse(table)
    info = plsc.get_sparse_core_info()
    nc, ns, lanes = info.num_cores, info.num_subcores, info.num_lanes
    sc = _make_sc_gather(n, vocab, c, nc, ns, lanes)
    logits, partials = sc(idx, tgt, table, table.reshape(-1), lse)
    loss = jnp.sum(partials) / n
    return (logits, loss)


# speedup vs baseline: 2.3068x; 1.1703x over previous
import functools

import jax
import jax.numpy as jnp
from jax import lax
from jax.experimental import pallas as pl
from jax.experimental.pallas import tpu as pltpu
from jax.experimental.pallas import tpu_sc as plsc


def _lse_block(x_ref, o_ref):
    x = x_ref[...]
    m = jnp.max(x, axis=1)
    s = jnp.sum(jnp.exp(x - m[:, None]), axis=1)
    o_ref[0, 0, :] = m + jnp.log(s)


def _row_lse(table):
    v, c = table.shape
    rows = 256
    grid = v // rows
    out = pl.pallas_call(
        _lse_block,
        grid=(grid,),
        in_specs=[pl.BlockSpec((rows, c), lambda i: (i, 0))],
        out_specs=pl.BlockSpec((1, 1, rows), lambda i: (i, 0, 0)),
        out_shape=jax.ShapeDtypeStruct((grid, 1, rows), jnp.float32),
    )(table)
    return out.reshape(-1)


def _make_sc_gather(n, vocab, c, nc, ns, lanes):
    nw = nc * ns
    npw = n // nw
    ch = lanes
    nch = npw // ch
    mesh = plsc.VectorSubcoreMesh(core_axis_name="c", subcore_axis_name="s")

    @functools.partial(
        pl.kernel,
        mesh=mesh,
        out_type=[
            jax.ShapeDtypeStruct((n, c), jnp.float32),
            jax.ShapeDtypeStruct((nw, lanes), jnp.float32),
        ],
        scratch_types=[
            pltpu.VMEM((npw,), jnp.int32),
            pltpu.VMEM((npw,), jnp.int32),
            pltpu.VMEM((vocab,), jnp.float32),
            pltpu.VMEM((ch, c), jnp.float32),
            pltpu.VMEM((lanes,), jnp.float32),
            pltpu.SemaphoreType.DMA,
        ],
        compiler_params=pltpu.CompilerParams(needs_layout_passes=False),
    )
    def sc_kernel(idx_hbm, tgt_hbm, table_hbm, lse_hbm, out_hbm, part_hbm,
                  idx_v, tgt_v, lse_v, rows_v, acc_v, sem):
        cid = lax.axis_index("c")
        sid = lax.axis_index("s")
        wid = sid * nc + cid
        base = wid * npw
        pltpu.sync_copy(idx_hbm.at[pl.ds(base, npw)], idx_v)
        pltpu.sync_copy(tgt_hbm.at[pl.ds(base, npw)], tgt_v)
        pltpu.sync_copy(lse_hbm, lse_v)
        lane = lax.iota(jnp.int32, lanes)

        def body(ci, acc):
            off = pl.multiple_of(ci * ch, 8)
            idx16 = idx_v[pl.ds(off, ch)]
            tgt16 = tgt_v[pl.ds(off, ch)]
            pltpu.async_copy(table_hbm.at[idx16], rows_v, sem).wait()
            pltpu.sync_copy(rows_v, out_hbm.at[pl.ds(base + off, ch)])
            tval16 = plsc.load_gather(rows_v, [lane, tgt16])
            lse16 = plsc.load_gather(lse_v, [idx16])
            return acc + (lse16 - tval16)

        acc = lax.fori_loop(0, nch, body, jnp.zeros((lanes,), jnp.float32))
        acc_v[...] = acc
        pltpu.sync_copy(acc_v, part_hbm.at[wid])

    return sc_kernel


def kernel(index, targets, table):
    b, t = index.shape
    vocab, c = table.shape
    n = b * t
    idx = index.reshape(-1).astype(jnp.int32)
    tgt = targets.reshape(-1).astype(jnp.int32)
    lse = _row_lse(table)
    info = plsc.get_sparse_core_info()
    nc, ns, lanes = info.num_cores, info.num_subcores, info.num_lanes
    sc = _make_sc_gather(n, vocab, c, nc, ns, lanes)
    logits, partials = sc(idx, tgt, table, lse)
    loss = jnp.sum(partials) / n
    return (logits, loss)


# speedup vs baseline: 2.3591x; 1.0227x over previous
import functools

import jax
import jax.numpy as jnp
from jax import lax
from jax.experimental import pallas as pl
from jax.experimental.pallas import tpu as pltpu
from jax.experimental.pallas import tpu_sc as plsc


def _lse_block(x_ref, o_ref):
    x = x_ref[...]
    m = jnp.max(x, axis=1)
    s = jnp.sum(jnp.exp(x - m[:, None]), axis=1)
    o_ref[0, 0, :] = m + jnp.log(s)


def _row_lse(table):
    v, c = table.shape
    rows = 256
    grid = v // rows
    out = pl.pallas_call(
        _lse_block,
        grid=(grid,),
        in_specs=[pl.BlockSpec((rows, c), lambda i: (i, 0))],
        out_specs=pl.BlockSpec((1, 1, rows), lambda i: (i, 0, 0)),
        out_shape=jax.ShapeDtypeStruct((grid, 1, rows), jnp.float32),
    )(table)
    return out.reshape(-1)


def _make_sc_gather(n, vocab, c, nc, ns, lanes):
    nw = nc * ns
    npw = n // nw
    ch = 8
    nch = npw // ch
    nbuf = 2
    gch = 128
    ngch = npw // gch
    ngrp = nch // nbuf
    mesh = plsc.VectorSubcoreMesh(core_axis_name="c", subcore_axis_name="s")

    @functools.partial(
        pl.kernel,
        mesh=mesh,
        out_type=[
            jax.ShapeDtypeStruct((n, c), jnp.float32),
            jax.ShapeDtypeStruct((nw, lanes), jnp.float32),
        ],
        scratch_types=[
            pltpu.VMEM((npw,), jnp.int32),
            pltpu.VMEM((npw + lanes,), jnp.int32),
            pltpu.VMEM((npw + lanes,), jnp.float32),
            [pltpu.VMEM((ch, c), jnp.float32) for _ in range(nbuf)],
            pltpu.VMEM((lanes,), jnp.float32),
            [pltpu.SemaphoreType.DMA for _ in range(nbuf)],
            [pltpu.SemaphoreType.DMA for _ in range(nbuf)],
            pltpu.SemaphoreType.DMA,
        ],
        compiler_params=pltpu.CompilerParams(needs_layout_passes=False),
    )
    def sc_kernel(idx_hbm, tgt_hbm, table_hbm, lse_hbm, out_hbm, part_hbm,
                  idx_v, tgt_v, lsev_v, bufs, acc_v, semg, sems, semx):
        cid = lax.axis_index("c")
        sid = lax.axis_index("s")
        wid = sid * nc + cid
        base = wid * npw
        pltpu.sync_copy(idx_hbm.at[pl.ds(base, npw)], idx_v)
        pltpu.sync_copy(tgt_hbm.at[pl.ds(base, npw)], tgt_v.at[pl.ds(0, npw)])
        lane = lax.iota(jnp.int32, lanes)
        zero16i = jnp.zeros((lanes,), jnp.int32)
        tgt_v[pl.ds(npw, lanes)] = zero16i

        for j in range(ngch):
            pltpu.async_copy(
                lse_hbm.at[idx_v.at[pl.ds(j * gch, gch)]],
                lsev_v.at[pl.ds(j * gch, gch)], semx)
        for j in range(ngch):
            pltpu.make_async_copy(
                lse_hbm.at[idx_v.at[pl.ds(j * gch, gch)]],
                lsev_v.at[pl.ds(j * gch, gch)], semx).wait()

        def start_gather(b, ci):
            off = pl.multiple_of(ci * ch, 8)
            pltpu.async_copy(
                table_hbm.at[idx_v.at[pl.ds(off, ch)]], bufs[b], semg[b])

        def wait_gather(b, ci):
            off = pl.multiple_of(ci * ch, 8)
            pltpu.make_async_copy(
                table_hbm.at[idx_v.at[pl.ds(off, ch)]], bufs[b],
                semg[b]).wait()

        def start_store(b, ci):
            off = pl.multiple_of(ci * ch, 8)
            pltpu.async_copy(bufs[b], out_hbm.at[pl.ds(base + off, ch)],
                             sems[b])

        def wait_store(b, ci):
            off = pl.multiple_of(ci * ch, 8)
            pltpu.make_async_copy(bufs[b], out_hbm.at[pl.ds(base + off, ch)],
                                  sems[b]).wait()

        def chunk_loss(b, ci, acc):
            off = pl.multiple_of(ci * ch, 8)
            tgt16 = tgt_v[pl.ds(off, lanes)]
            lse16 = lsev_v[pl.ds(off, lanes)]
            row16 = jnp.bitwise_and(lane, ch - 1)
            tval16 = plsc.load_gather(bufs[b], [row16, tgt16])
            return acc + jnp.where(lane < ch, lse16 - tval16, 0.0)

        for b in range(nbuf):
            start_gather(b, b)

        def body(k, acc):
            for b in range(nbuf):
                ci = k * nbuf + b
                wait_gather(b, ci)
                acc = chunk_loss(b, ci, acc)
                start_store(b, ci)
            for b in range(nbuf):
                ci = k * nbuf + b

                @pl.when(ci + nbuf < nch)
                def _():
                    wait_store(b, ci)
                    start_gather(b, ci + nbuf)
            return acc

        acc = lax.fori_loop(0, ngrp, body, jnp.zeros((lanes,), jnp.float32))
        for b in range(nbuf):
            wait_store(b, nch - nbuf + b)
        acc_v[...] = acc
        pltpu.sync_copy(acc_v, part_hbm.at[wid])

    return sc_kernel


def kernel(index, targets, table):
    b, t = index.shape
    vocab, c = table.shape
    n = b * t
    idx = index.reshape(-1).astype(jnp.int32)
    tgt = targets.reshape(-1).astype(jnp.int32)
    lse = _row_lse(table)
    info = plsc.get_sparse_core_info()
    nc, ns, lanes = info.num_cores, info.num_subcores, info.num_lanes
    sc = _make_sc_gather(n, vocab, c, nc, ns, lanes)
    logits, partials = sc(idx, tgt, table, lse)
    loss = jnp.sum(partials) / n
    return (logits, loss)


# speedup vs baseline: 2.4094x; 1.0213x over previous
import functools

import jax
import jax.numpy as jnp
from jax import lax
from jax.experimental import pallas as pl
from jax.experimental.pallas import tpu as pltpu
from jax.experimental.pallas import tpu_sc as plsc


def _lse_block(x_ref, o_ref):
    x = x_ref[...]
    m = jnp.max(x, axis=1)
    s = jnp.sum(jnp.exp(x - m[:, None]), axis=1)
    o_ref[0, 0, :] = m + jnp.log(s)


def _row_lse(table):
    v, c = table.shape
    rows = 256
    grid = v // rows
    out = pl.pallas_call(
        _lse_block,
        grid=(grid,),
        in_specs=[pl.BlockSpec((rows, c), lambda i: (i, 0))],
        out_specs=pl.BlockSpec((1, 1, rows), lambda i: (i, 0, 0)),
        out_shape=jax.ShapeDtypeStruct((grid, 1, rows), jnp.float32),
    )(table)
    return out.reshape(-1)


def _make_sc_gather(n, vocab, c, nc, ns, lanes):
    nw = nc * ns
    npw = n // nw
    ch = 8
    nch = npw // ch
    nbuf = 3
    gch = 128
    ngch = npw // gch
    ngrp = nch // nbuf
    nrem = nch - ngrp * nbuf
    mesh = plsc.VectorSubcoreMesh(core_axis_name="c", subcore_axis_name="s")

    @functools.partial(
        pl.kernel,
        mesh=mesh,
        out_type=[
            jax.ShapeDtypeStruct((n, c), jnp.float32),
            jax.ShapeDtypeStruct((nw, lanes), jnp.float32),
        ],
        scratch_types=[
            pltpu.VMEM((npw,), jnp.int32),
            pltpu.VMEM((npw + lanes,), jnp.int32),
            pltpu.VMEM((npw + lanes,), jnp.float32),
            [pltpu.VMEM((ch, c), jnp.float32) for _ in range(nbuf)],
            pltpu.VMEM((lanes,), jnp.float32),
            [pltpu.SemaphoreType.DMA for _ in range(nbuf)],
            [pltpu.SemaphoreType.DMA for _ in range(nbuf)],
            pltpu.SemaphoreType.DMA,
        ],
        compiler_params=pltpu.CompilerParams(needs_layout_passes=False),
    )
    def sc_kernel(idx_hbm, tgt_hbm, table_hbm, lse_hbm, out_hbm, part_hbm,
                  idx_v, tgt_v, lsev_v, bufs, acc_v, semg, sems, semx):
        cid = lax.axis_index("c")
        sid = lax.axis_index("s")
        wid = sid * nc + cid
        base = wid * npw
        pltpu.sync_copy(idx_hbm.at[pl.ds(base, npw)], idx_v)
        pltpu.sync_copy(tgt_hbm.at[pl.ds(base, npw)], tgt_v.at[pl.ds(0, npw)])
        lane = lax.iota(jnp.int32, lanes)
        zero16i = jnp.zeros((lanes,), jnp.int32)
        tgt_v[pl.ds(npw, lanes)] = zero16i

        for j in range(ngch):
            pltpu.async_copy(
                lse_hbm.at[idx_v.at[pl.ds(j * gch, gch)]],
                lsev_v.at[pl.ds(j * gch, gch)], semx)
        for j in range(ngch):
            pltpu.make_async_copy(
                lse_hbm.at[idx_v.at[pl.ds(j * gch, gch)]],
                lsev_v.at[pl.ds(j * gch, gch)], semx).wait()

        def start_gather(b, ci):
            off = pl.multiple_of(ci * ch, 8)
            pltpu.async_copy(
                table_hbm.at[idx_v.at[pl.ds(off, ch)]], bufs[b], semg[b])

        def wait_gather(b, ci):
            off = pl.multiple_of(ci * ch, 8)
            pltpu.make_async_copy(
                table_hbm.at[idx_v.at[pl.ds(off, ch)]], bufs[b],
                semg[b]).wait()

        def start_store(b, ci):
            off = pl.multiple_of(ci * ch, 8)
            pltpu.async_copy(bufs[b], out_hbm.at[pl.ds(base + off, ch)],
                             sems[b])

        def wait_store(b, ci):
            off = pl.multiple_of(ci * ch, 8)
            pltpu.make_async_copy(bufs[b], out_hbm.at[pl.ds(base + off, ch)],
                                  sems[b]).wait()

        def chunk_loss(b, ci, acc):
            off = pl.multiple_of(ci * ch, 8)
            tgt16 = tgt_v[pl.ds(off, lanes)]
            lse16 = lsev_v[pl.ds(off, lanes)]
            row16 = jnp.bitwise_and(lane, ch - 1)
            tval16 = plsc.load_gather(bufs[b], [row16, tgt16])
            return acc + jnp.where(lane < ch, lse16 - tval16, 0.0)

        for b in range(nbuf):
            start_gather(b, b)

        def body(k, acc):
            for b in range(nbuf):
                ci = k * nbuf + b
                wait_gather(b, ci)
                acc = chunk_loss(b, ci, acc)
                start_store(b, ci)
            for b in range(nbuf):
                ci = k * nbuf + b

                @pl.when(ci + nbuf < nch)
                def _():
                    wait_store(b, ci)
                    start_gather(b, ci + nbuf)
            return acc

        acc = lax.fori_loop(0, ngrp, body, jnp.zeros((lanes,), jnp.float32))
        for r in range(nrem):
            ci = ngrp * nbuf + r
            wait_gather(r, ci)
            acc = chunk_loss(r, ci, acc)
            start_store(r, ci)
        for b in range(nbuf):
            if b < nrem:
                wait_store(b, ngrp * nbuf + b)
            else:
                wait_store(b, (ngrp - 1) * nbuf + b)
        acc_v[...] = acc
        pltpu.sync_copy(acc_v, part_hbm.at[wid])

    return sc_kernel


def kernel(index, targets, table):
    b, t = index.shape
    vocab, c = table.shape
    n = b * t
    idx = index.reshape(-1).astype(jnp.int32)
    tgt = targets.reshape(-1).astype(jnp.int32)
    lse = _row_lse(table)
    info = plsc.get_sparse_core_info()
    nc, ns, lanes = info.num_cores, info.num_subcores, info.num_lanes
    sc = _make_sc_gather(n, vocab, c, nc, ns, lanes)
    logits, partials = sc(idx, tgt, table, lse)
    loss = jnp.sum(partials) / n
    return (logits, loss)
